# baseline (speedup 1.0000x reference)
import jax
import jax.numpy as jnp
from jax.experimental import pallas as pl
from jax.experimental.pallas import tpu as pltpu

N = 4096
C = 16
RAD = 256
RAU = 128
RB = 512

_F32 = jnp.float32
_BF16 = jnp.bfloat16


def _dot(a, b):
    return jnp.dot(a, b, preferred_element_type=_F32)


def _mega_body(x_ref, g1_ref, v2_ref, b2_ref, ld_hbm, lu_hbm,
               out_ref, lub_hbm,
               ldb_v, xd1_v, xu1_v, h_v, hd1_v, hu1_v,
               p_v, xd1b_v, xu1b_v):
    x = x_ref[...]

    xb = x.astype(_BF16)

    def ad_body(ld_ref):
        i = pl.program_id(0)
        rows = pl.ds(i * RAD, RAD)
        ld = ld_ref[...].astype(_BF16)
        xd1_v[rows, :] = _dot(ld, xb)
        ldb_v[rows, :] = ld

    pltpu.emit_pipeline(
        ad_body,
        grid=(N // RAD,),
        in_specs=[pl.BlockSpec((RAD, N), lambda i: (i, 0))],
    )(ld_hbm)

    def au_body(lu_ref, lub_ref):
        i = pl.program_id(0)
        rows = pl.ds(i * RAU, RAU)
        lu = lu_ref[...].astype(_BF16)
        xu1_v[rows, :] = _dot(lu, xb)
        lub_ref[...] = lu

    pltpu.emit_pipeline(
        au_body,
        grid=(N // RAU,),
        in_specs=[pl.BlockSpec((RAU, N), lambda i: (i, 0))],
        out_specs=[pl.BlockSpec((RAU, N), lambda i: (i, 0))],
    )(lu_hbm, lub_hbm)

    p_v[...] = (_dot(x, g1_ref[0]) + _dot(xd1_v[...], g1_ref[1])
                + _dot(xu1_v[...], g1_ref[3]))
    xd1b_v[...] = xd1_v[...].astype(_BF16)
    xu1b_v[...] = xu1_v[...].astype(_BF16)

    def b_body(lub_ref):
        i = pl.program_id(0)
        rows = pl.ds(i * RB, RB)
        xd2 = _dot(ldb_v[rows, :], xd1b_v[...])
        xu2 = _dot(lub_ref[...], xu1b_v[...])
        h_v[rows, :] = p_v[rows, :] + _dot(xd2, g1_ref[2]) + _dot(xu2, g1_ref[4])

    lub_spec = [pl.BlockSpec((RB, N), lambda i: (i, 0))]
    pltpu.emit_pipeline(b_body, grid=(N // RB,), in_specs=lub_spec)(lub_hbm)

    hb_v = xd1b_v
    hb_v[...] = h_v[...].astype(_BF16)

    def c_body(lub_ref):
        i = pl.program_id(0)
        rows = pl.ds(i * RB, RB)
        hd1_v[rows, :] = _dot(ldb_v[rows, :], hb_v[...])
        hu1_v[rows, :] = _dot(lub_ref[...], hb_v[...])

    pltpu.emit_pipeline(c_body, grid=(N // RB,), in_specs=lub_spec)(lub_hbm)

    p_v[...] = (_dot(h_v[...], v2_ref[0]) + _dot(hd1_v[...], v2_ref[1])
                + _dot(hu1_v[...], v2_ref[3]) + b2_ref[...])
    hd1b_v = xu1b_v
    hd1b_v[...] = hd1_v[...].astype(_BF16)
    hu1b_v = hb_v
    hu1b_v[...] = hu1_v[...].astype(_BF16)

    def d_body(lub_ref):
        i = pl.program_id(0)
        rows = pl.ds(i * RB, RB)
        hd2 = _dot(ldb_v[rows, :], hd1b_v[...])
        hu2 = _dot(lub_ref[...], hu1b_v[...])
        out_ref[rows, :] = (p_v[rows, :] + _dot(hd2, v2_ref[2])
                            + _dot(hu2, v2_ref[4]))

    pltpu.emit_pipeline(d_body, grid=(N // RB,), in_specs=lub_spec)(lub_hbm)


def kernel(x, laplacian_down, laplacian_up, W1, W2, W_lin, b_lin):
    G1 = jnp.transpose(W1, (2, 0, 1))
    V2 = jnp.einsum("iok,oj->kij", W2, W_lin)
    b2 = b_lin.reshape(1, C).astype(_F32)

    out, _ = pl.pallas_call(
        _mega_body,
        in_specs=[
            pl.BlockSpec(memory_space=pltpu.MemorySpace.VMEM),
            pl.BlockSpec(memory_space=pltpu.MemorySpace.VMEM),
            pl.BlockSpec(memory_space=pltpu.MemorySpace.VMEM),
            pl.BlockSpec(memory_space=pltpu.MemorySpace.VMEM),
            pl.BlockSpec(memory_space=pltpu.MemorySpace.HBM),
            pl.BlockSpec(memory_space=pltpu.MemorySpace.HBM),
        ],
        out_specs=[
            pl.BlockSpec(memory_space=pltpu.MemorySpace.VMEM),
            pl.BlockSpec(memory_space=pltpu.MemorySpace.HBM),
        ],
        out_shape=[
            jax.ShapeDtypeStruct((N, C), _F32),
            jax.ShapeDtypeStruct((N, N), _BF16),
        ],
        compiler_params=pltpu.CompilerParams(
            vmem_limit_bytes=64 * 1024 * 1024),
        scratch_shapes=[
            pltpu.VMEM((N, N), _BF16),
            pltpu.VMEM((N, C), _F32),
            pltpu.VMEM((N, C), _F32),
            pltpu.VMEM((N, C), _F32),
            pltpu.VMEM((N, C), _F32),
            pltpu.VMEM((N, C), _F32),
            pltpu.VMEM((N, C), _F32),
            pltpu.VMEM((N, C), _BF16),
            pltpu.VMEM((N, C), _BF16),
        ],
    )(x, G1, V2, b2, laplacian_down, laplacian_up)
    return out


# speedup vs baseline: 1.0739x; 1.0739x over previous
import jax
import jax.numpy as jnp
from jax.experimental import pallas as pl
from jax.experimental.pallas import tpu as pltpu

N = 4096
C = 16
RA = 128
RB = 512

_F32 = jnp.float32
_BF16 = jnp.bfloat16


def _dot(a, b):
    return jnp.dot(a, b, preferred_element_type=_F32)


def _mega_body(x_ref, g1_ref, v2_ref, b2_ref, ld_hbm, lu_hbm,
               out_ref, lub_hbm,
               ldb_v, xd1_v, xu1_v, h_v, hd1_v, hu1_v,
               p_v, xd1b_v, xu1b_v):
    x = x_ref[...]

    xb = x.astype(_BF16)

    def a_body(ld_ref, lu_ref, lub_ref):
        i = pl.program_id(0)
        rows = pl.ds(i * RA, RA)
        ld = ld_ref[...].astype(_BF16)
        lu = lu_ref[...].astype(_BF16)
        xd1_v[rows, :] = _dot(ld, xb)
        xu1_v[rows, :] = _dot(lu, xb)
        ldb_v[rows, :] = ld
        lub_ref[...] = lu

    pltpu.emit_pipeline(
        a_body,
        grid=(N // RA,),
        in_specs=[
            pl.BlockSpec((RA, N), lambda i: (i, 0)),
            pl.BlockSpec((RA, N), lambda i: (i, 0)),
        ],
        out_specs=[pl.BlockSpec((RA, N), lambda i: (i, 0))],
    )(ld_hbm, lu_hbm, lub_hbm)

    p_v[...] = (_dot(x, g1_ref[0]) + _dot(xd1_v[...], g1_ref[1])
                + _dot(xu1_v[...], g1_ref[3]))
    xd1b_v[...] = xd1_v[...].astype(_BF16)
    xu1b_v[...] = xu1_v[...].astype(_BF16)

    def b_body(lub_ref):
        i = pl.program_id(0)
        rows = pl.ds(i * RB, RB)
        xd2 = _dot(ldb_v[rows, :], xd1b_v[...])
        xu2 = _dot(lub_ref[...], xu1b_v[...])
        h_v[rows, :] = p_v[rows, :] + _dot(xd2, g1_ref[2]) + _dot(xu2, g1_ref[4])

    lub_spec = [pl.BlockSpec((RB, N), lambda i: (i, 0))]
    pltpu.emit_pipeline(b_body, grid=(N // RB,), in_specs=lub_spec)(lub_hbm)

    hb_v = xd1b_v
    hb_v[...] = h_v[...].astype(_BF16)

    def c_body(lub_ref):
        i = pl.program_id(0)
        rows = pl.ds(i * RB, RB)
        hd1_v[rows, :] = _dot(ldb_v[rows, :], hb_v[...])
        hu1_v[rows, :] = _dot(lub_ref[...], hb_v[...])

    pltpu.emit_pipeline(c_body, grid=(N // RB,), in_specs=lub_spec)(lub_hbm)

    p_v[...] = (_dot(h_v[...], v2_ref[0]) + _dot(hd1_v[...], v2_ref[1])
                + _dot(hu1_v[...], v2_ref[3]) + b2_ref[...])
    hd1b_v = xu1b_v
    hd1b_v[...] = hd1_v[...].astype(_BF16)
    hu1b_v = hb_v
    hu1b_v[...] = hu1_v[...].astype(_BF16)

    def d_body(lub_ref):
        i = pl.program_id(0)
        rows = pl.ds(i * RB, RB)
        hd2 = _dot(ldb_v[rows, :], hd1b_v[...])
        hu2 = _dot(lub_ref[...], hu1b_v[...])
        out_ref[rows, :] = (p_v[rows, :] + _dot(hd2, v2_ref[2])
                            + _dot(hu2, v2_ref[4]))

    pltpu.emit_pipeline(d_body, grid=(N // RB,), in_specs=lub_spec)(lub_hbm)


def kernel(x, laplacian_down, laplacian_up, W1, W2, W_lin, b_lin):
    G1 = jnp.transpose(W1, (2, 0, 1))
    V2 = jnp.einsum("iok,oj->kij", W2, W_lin)
    b2 = b_lin.reshape(1, C).astype(_F32)

    out, _ = pl.pallas_call(
        _mega_body,
        in_specs=[
            pl.BlockSpec(memory_space=pltpu.MemorySpace.VMEM),
            pl.BlockSpec(memory_space=pltpu.MemorySpace.VMEM),
            pl.BlockSpec(memory_space=pltpu.MemorySpace.VMEM),
            pl.BlockSpec(memory_space=pltpu.MemorySpace.VMEM),
            pl.BlockSpec(memory_space=pltpu.MemorySpace.HBM),
            pl.BlockSpec(memory_space=pltpu.MemorySpace.HBM),
        ],
        out_specs=[
            pl.BlockSpec(memory_space=pltpu.MemorySpace.VMEM),
            pl.BlockSpec(memory_space=pltpu.MemorySpace.HBM),
        ],
        out_shape=[
            jax.ShapeDtypeStruct((N, C), _F32),
            jax.ShapeDtypeStruct((N, N), _BF16),
        ],
        compiler_params=pltpu.CompilerParams(
            vmem_limit_bytes=64 * 1024 * 1024),
        scratch_shapes=[
            pltpu.VMEM((N, N), _BF16),
            pltpu.VMEM((N, C), _F32),
            pltpu.VMEM((N, C), _F32),
            pltpu.VMEM((N, C), _F32),
            pltpu.VMEM((N, C), _F32),
            pltpu.VMEM((N, C), _F32),
            pltpu.VMEM((N, C), _F32),
            pltpu.VMEM((N, C), _BF16),
            pltpu.VMEM((N, C), _BF16),
        ],
    )(x, G1, V2, b2, laplacian_down, laplacian_up)
    return out
